# Initial kernel scaffold; baseline (speedup 1.0000x reference)
import jax
import jax.numpy as jnp
from jax.experimental import pallas as pl


def kernel(x, current_uncertainty, rW1, rb1, rW2, rb2, rW3, rb3, W0, b0, W1, b1, W2, b2, W3, b3):
    raise NotImplementedError("write your pallas kernel here")


# speedup vs baseline: 1.3041x; 1.3041x over previous
import functools

import jax
import jax.numpy as jnp
from jax.experimental import pallas as pl
from jax.experimental.pallas import tpu as pltpu

N = 32768
D = 768
BLK = 256
NBLK = N // BLK


def _router_body(u_ref, w1_ref, b1_ref, w2_ref, b2_ref, w3_ref, b3_ref,
                 mask_ref, flags_ref):
    u = u_ref[...]
    umin = jnp.min(u)
    umax = jnp.max(u)
    un = (u - umin) / (umax - umin + 1e-8)
    acc = [b2_ref[0, k] * jnp.ones_like(un) for k in range(16)]
    for j in range(32):
        hj = jax.nn.relu(un * w1_ref[0, j] + b1_ref[0, j])
        for k in range(16):
            acc[k] = acc[k] + hj * w2_ref[j, k]
    l0 = jnp.full_like(un, b3_ref[0, 0])
    l1 = jnp.full_like(un, b3_ref[0, 1])
    l2 = jnp.full_like(un, b3_ref[0, 2])
    for k in range(16):
        hk = jax.nn.relu(acc[k])
        l0 = l0 + hk * w3_ref[k, 0]
        l1 = l1 + hk * w3_ref[k, 1]
        l2 = l2 + hk * w3_ref[k, 2]
    d = jnp.where((l1 > l0) & (l1 >= l2), 1.0,
                  jnp.where((l2 > l0) & (l2 > l1), 2.0, 0.0))
    mask_ref[...] = d
    flags_ref[...] = jnp.max(d, axis=1, keepdims=True)


def _chain_body(flags_ref, x_ref, m_ref,
                w0_ref, b0_ref, w1_ref, b1_ref, w2_ref, b2_ref, w3_ref, b3_ref,
                out_ref):
    i = pl.program_id(0)
    f = flags_ref[i]
    x = x_ref[...]
    s1 = jnp.dot(x, w0_ref[...], preferred_element_type=jnp.float32) + b0_ref[...]
    m = m_ref[...]

    @pl.when(f == 0)
    def _():
        out_ref[...] = s1

    @pl.when(f >= 1)
    def _():
        s2 = jnp.dot(s1, w1_ref[...], preferred_element_type=jnp.float32) + b1_ref[...]

        @pl.when(f == 1)
        def _():
            out_ref[...] = jnp.where(m == 0.0, s1, s2)

        @pl.when(f == 2)
        def _():
            s3 = jnp.dot(s2, w2_ref[...], preferred_element_type=jnp.float32) + b2_ref[...]
            s4 = jnp.dot(s3, w3_ref[...], preferred_element_type=jnp.float32) + b3_ref[...]
            out_ref[...] = jnp.where(m == 0.0, s1, jnp.where(m == 1.0, s2, s4))


_INTERPRET = False


def _full(shape):
    return pl.BlockSpec(shape, lambda i, flags: (0, 0))


def kernel(x, current_uncertainty, rW1, rb1, rW2, rb2, rW3, rb3,
           W0, b0, W1, b1, W2, b2, W3, b3):
    u2 = current_uncertainty.reshape(NBLK, BLK)
    mask2, flags2 = pl.pallas_call(
        _router_body,
        out_shape=(jax.ShapeDtypeStruct((NBLK, BLK), jnp.float32),
                   jax.ShapeDtypeStruct((NBLK, 1), jnp.float32)),
        interpret=_INTERPRET,
    )(u2, rW1, rb1.reshape(1, 32), rW2, rb2.reshape(1, 16), rW3,
      rb3.reshape(1, 3))

    flags = flags2.reshape(NBLK).astype(jnp.int32)
    mask = mask2.reshape(N)

    grid_spec = pltpu.PrefetchScalarGridSpec(
        num_scalar_prefetch=1,
        grid=(NBLK,),
        in_specs=[
            pl.BlockSpec((BLK, D), lambda i, flags: (i, 0)),
            pl.BlockSpec((BLK, 1), lambda i, flags: (i, 0)),
            _full((D, D)), _full((1, D)),
            _full((D, D)), _full((1, D)),
            _full((D, D)), _full((1, D)),
            _full((D, D)), _full((1, D)),
        ],
        out_specs=pl.BlockSpec((BLK, D), lambda i, flags: (i, 0)),
    )
    out = pl.pallas_call(
        _chain_body,
        grid_spec=grid_spec,
        out_shape=jax.ShapeDtypeStruct((N, D), jnp.float32),
        interpret=_INTERPRET,
    )(flags, x, mask.reshape(N, 1),
      W0, b0.reshape(1, D), W1, b1.reshape(1, D),
      W2, b2.reshape(1, D), W3, b3.reshape(1, D))
    return out, mask


# speedup vs baseline: 1.6967x; 1.3010x over previous
import functools

import jax
import jax.numpy as jnp
from jax.experimental import pallas as pl
from jax.experimental.pallas import tpu as pltpu

N = 32768
D = 768
BLK = 512
NBLK = N // BLK


def _router_body(u_ref, w1_ref, b1_ref, w2_ref, b2_ref, w3_ref, b3_ref,
                 mask_ref, flags_ref):
    u = u_ref[...]
    umin = jnp.min(u)
    umax = jnp.max(u)
    un = (u - umin) / (umax - umin + 1e-8)
    acc = [b2_ref[0, k] * jnp.ones_like(un) for k in range(16)]
    for j in range(32):
        hj = jax.nn.relu(un * w1_ref[0, j] + b1_ref[0, j])
        for k in range(16):
            acc[k] = acc[k] + hj * w2_ref[j, k]
    l0 = jnp.full_like(un, b3_ref[0, 0])
    l1 = jnp.full_like(un, b3_ref[0, 1])
    l2 = jnp.full_like(un, b3_ref[0, 2])
    for k in range(16):
        hk = jax.nn.relu(acc[k])
        l0 = l0 + hk * w3_ref[k, 0]
        l1 = l1 + hk * w3_ref[k, 1]
        l2 = l2 + hk * w3_ref[k, 2]
    d = jnp.where((l1 > l0) & (l1 >= l2), 1.0,
                  jnp.where((l2 > l0) & (l2 > l1), 2.0, 0.0))
    mask_ref[...] = d
    flags_ref[...] = jnp.max(d, axis=1, keepdims=True)


def _chain_body(flags_ref, x_ref, m_ref,
                w0_ref, b0_ref, w1_ref, b1_ref, w2_ref, b2_ref, w3_ref, b3_ref,
                out_ref):
    i = pl.program_id(0)
    f = flags_ref[i]
    x = x_ref[...]
    s1 = jnp.dot(x, w0_ref[...], preferred_element_type=jnp.float32) + b0_ref[...]
    m = m_ref[...]

    @pl.when(f == 0)
    def _():
        out_ref[...] = s1

    @pl.when(f >= 1)
    def _():
        s2 = jnp.dot(s1, w1_ref[...], preferred_element_type=jnp.float32) + b1_ref[...]

        @pl.when(f == 1)
        def _():
            out_ref[...] = jnp.where(m == 0.0, s1, s2)

        @pl.when(f == 2)
        def _():
            s3 = jnp.dot(s2, w2_ref[...], preferred_element_type=jnp.float32) + b2_ref[...]
            s4 = jnp.dot(s3, w3_ref[...], preferred_element_type=jnp.float32) + b3_ref[...]
            out_ref[...] = jnp.where(m == 0.0, s1, jnp.where(m == 1.0, s2, s4))


_INTERPRET = False


def _full(shape):
    return pl.BlockSpec(shape, lambda i, flags: (0, 0))


def kernel(x, current_uncertainty, rW1, rb1, rW2, rb2, rW3, rb3,
           W0, b0, W1, b1, W2, b2, W3, b3):
    u2 = current_uncertainty.reshape(NBLK, BLK)
    mask2, flags2 = pl.pallas_call(
        _router_body,
        out_shape=(jax.ShapeDtypeStruct((NBLK, BLK), jnp.float32),
                   jax.ShapeDtypeStruct((NBLK, 1), jnp.float32)),
        interpret=_INTERPRET,
    )(u2, rW1, rb1.reshape(1, 32), rW2, rb2.reshape(1, 16), rW3,
      rb3.reshape(1, 3))

    flags = flags2.reshape(NBLK).astype(jnp.int32)
    mask = mask2.reshape(N)

    grid_spec = pltpu.PrefetchScalarGridSpec(
        num_scalar_prefetch=1,
        grid=(NBLK,),
        in_specs=[
            pl.BlockSpec((BLK, D), lambda i, flags: (i, 0)),
            pl.BlockSpec((BLK, 1), lambda i, flags: (i, 0)),
            _full((D, D)), _full((1, D)),
            _full((D, D)), _full((1, D)),
            _full((D, D)), _full((1, D)),
            _full((D, D)), _full((1, D)),
        ],
        out_specs=pl.BlockSpec((BLK, D), lambda i, flags: (i, 0)),
    )
    out = pl.pallas_call(
        _chain_body,
        grid_spec=grid_spec,
        out_shape=jax.ShapeDtypeStruct((N, D), jnp.float32),
        interpret=_INTERPRET,
    )(flags, x, mask.reshape(N, 1),
      W0, b0.reshape(1, D), W1, b1.reshape(1, D),
      W2, b2.reshape(1, D), W3, b3.reshape(1, D))
    return out, mask


# speedup vs baseline: 1.9877x; 1.1715x over previous
import functools

import jax
import jax.numpy as jnp
from jax.experimental import pallas as pl
from jax.experimental.pallas import tpu as pltpu

N = 32768
D = 768
BLK = 1024
NBLK = N // BLK


def _router_body(u_ref, w1_ref, b1_ref, w2_ref, b2_ref, w3_ref, b3_ref,
                 mask_ref, flags_ref):
    u = u_ref[...]
    umin = jnp.min(u)
    umax = jnp.max(u)
    un = (u - umin) / (umax - umin + 1e-8)
    acc = [b2_ref[0, k] * jnp.ones_like(un) for k in range(16)]
    for j in range(32):
        hj = jax.nn.relu(un * w1_ref[0, j] + b1_ref[0, j])
        for k in range(16):
            acc[k] = acc[k] + hj * w2_ref[j, k]
    l0 = jnp.full_like(un, b3_ref[0, 0])
    l1 = jnp.full_like(un, b3_ref[0, 1])
    l2 = jnp.full_like(un, b3_ref[0, 2])
    for k in range(16):
        hk = jax.nn.relu(acc[k])
        l0 = l0 + hk * w3_ref[k, 0]
        l1 = l1 + hk * w3_ref[k, 1]
        l2 = l2 + hk * w3_ref[k, 2]
    d = jnp.where((l1 > l0) & (l1 >= l2), 1.0,
                  jnp.where((l2 > l0) & (l2 > l1), 2.0, 0.0))
    mask_ref[...] = d
    flags_ref[...] = jnp.max(d, axis=1, keepdims=True)


def _chain_body(flags_ref, x_ref, m_ref,
                w0_ref, b0_ref, w1_ref, b1_ref, w2_ref, b2_ref, w3_ref, b3_ref,
                out_ref):
    i = pl.program_id(0)
    f = flags_ref[i]
    x = x_ref[...]
    s1 = jnp.dot(x, w0_ref[...], preferred_element_type=jnp.float32) + b0_ref[...]
    m = m_ref[...]

    @pl.when(f == 0)
    def _():
        out_ref[...] = s1

    @pl.when(f >= 1)
    def _():
        s2 = jnp.dot(s1, w1_ref[...], preferred_element_type=jnp.float32) + b1_ref[...]

        @pl.when(f == 1)
        def _():
            out_ref[...] = jnp.where(m == 0.0, s1, s2)

        @pl.when(f == 2)
        def _():
            s3 = jnp.dot(s2, w2_ref[...], preferred_element_type=jnp.float32) + b2_ref[...]
            s4 = jnp.dot(s3, w3_ref[...], preferred_element_type=jnp.float32) + b3_ref[...]
            out_ref[...] = jnp.where(m == 0.0, s1, jnp.where(m == 1.0, s2, s4))


_INTERPRET = False


def _full(shape):
    return pl.BlockSpec(shape, lambda i, flags: (0, 0))


def kernel(x, current_uncertainty, rW1, rb1, rW2, rb2, rW3, rb3,
           W0, b0, W1, b1, W2, b2, W3, b3):
    u2 = current_uncertainty.reshape(NBLK, BLK)
    mask2, flags2 = pl.pallas_call(
        _router_body,
        out_shape=(jax.ShapeDtypeStruct((NBLK, BLK), jnp.float32),
                   jax.ShapeDtypeStruct((NBLK, 1), jnp.float32)),
        interpret=_INTERPRET,
    )(u2, rW1, rb1.reshape(1, 32), rW2, rb2.reshape(1, 16), rW3,
      rb3.reshape(1, 3))

    flags = flags2.reshape(NBLK).astype(jnp.int32)
    mask = mask2.reshape(N)

    grid_spec = pltpu.PrefetchScalarGridSpec(
        num_scalar_prefetch=1,
        grid=(NBLK,),
        in_specs=[
            pl.BlockSpec((BLK, D), lambda i, flags: (i, 0)),
            pl.BlockSpec((BLK, 1), lambda i, flags: (i, 0)),
            _full((D, D)), _full((1, D)),
            _full((D, D)), _full((1, D)),
            _full((D, D)), _full((1, D)),
            _full((D, D)), _full((1, D)),
        ],
        out_specs=pl.BlockSpec((BLK, D), lambda i, flags: (i, 0)),
    )
    out = pl.pallas_call(
        _chain_body,
        grid_spec=grid_spec,
        out_shape=jax.ShapeDtypeStruct((N, D), jnp.float32),
        interpret=_INTERPRET,
    )(flags, x, mask.reshape(N, 1),
      W0, b0.reshape(1, D), W1, b1.reshape(1, D),
      W2, b2.reshape(1, D), W3, b3.reshape(1, D))
    return out, mask
